# Initial kernel scaffold; baseline (speedup 1.0000x reference)
import functools

import jax
import jax.numpy as jnp
from jax import lax
from jax.experimental import pallas as pl
from jax.experimental.pallas import tpu as pltpu
from jax.experimental.pallas import tpu_sc as plsc

N = 10000
E = 320000
D_IN = 128
DIM = 256
G = 128

NC = 2
NS = 16
NW = NC * NS
CHUNK = 128
CHUNKS_PER_W = 80
E_PAD = NW * CHUNKS_PER_W * CHUNK
N_PAD = N + 8
ROWS_PER_S = N // NS

BLK = 256
N_TC = 10240
NB = N_TC // BLK
BN_INV = 1.0 / (1.0 + 1e-5) ** 0.5


_sc_mesh = plsc.VectorSubcoreMesh(core_axis_name="c", subcore_axis_name="s")


@functools.partial(
    pl.kernel,
    out_type=jax.ShapeDtypeStruct((NC, N, D_IN), jnp.float32),
    mesh=_sc_mesh,
    scratch_types=[
        pltpu.VMEM((CHUNKS_PER_W, CHUNK), jnp.int32),
        pltpu.VMEM((CHUNKS_PER_W, CHUNK), jnp.int32),
        pltpu.VMEM((CHUNK, D_IN), jnp.float32),
        pltpu.VMEM((CHUNK, D_IN), jnp.float32),
        pltpu.VMEM_SHARED((N_PAD, D_IN), jnp.float32),
        pltpu.SemaphoreType.DMA,
        pltpu.SemaphoreType.DMA,
    ],
)
def _sc_agg(x_hbm, src_hbm, dst_hbm, out_hbm,
            src_v, dst_v, rows0, rows1, acc, sem0, sem1):
    c = lax.axis_index("c")
    s = lax.axis_index("s")
    w = s * NC + c
    pltpu.sync_copy(src_hbm.at[w], src_v)
    pltpu.sync_copy(dst_hbm.at[w], dst_v)
    row0 = s * ROWS_PER_S
    pltpu.sync_copy(x_hbm.at[pl.ds(row0, ROWS_PER_S)],
                    acc.at[pl.ds(row0, ROWS_PER_S)])
    plsc.subcore_barrier()

    def pair(i, carry):
        j0 = 2 * i
        j1 = j0 + 1
        g0 = pltpu.async_copy(x_hbm.at[src_v.at[j0]], rows0, sem0)
        g1 = pltpu.async_copy(x_hbm.at[src_v.at[j1]], rows1, sem1)
        g0.wait()
        pltpu.sync_copy(rows0, acc.at[dst_v.at[j0]], add=True)
        g1.wait()
        pltpu.sync_copy(rows1, acc.at[dst_v.at[j1]], add=True)
        return carry

    lax.fori_loop(0, CHUNKS_PER_W // 2, pair, 0)
    plsc.subcore_barrier()
    pltpu.sync_copy(acc.at[pl.ds(row0, ROWS_PER_S)],
                    out_hbm.at[c, pl.ds(row0, ROWS_PER_S)])


def _tc_head_body(x_ref, a_ref, b_ref, batch_ref,
                  w1_ref, b1_ref, w2_ref, b2_ref, bng_ref, bnb_ref,
                  fc1w_ref, fc1b_ref, fc2w_ref, fc2b_ref,
                  out_ref, acc_ref):
    i = pl.program_id(0)

    @pl.when(i == 0)
    def _():
        acc_ref[...] = jnp.zeros_like(acc_ref)

    h = a_ref[...] + b_ref[...] - x_ref[...]
    h = jnp.maximum(
        jax.lax.dot(h, w1_ref[...], preferred_element_type=jnp.float32)
        + b1_ref[...], 0.0)
    h = jax.lax.dot(h, w2_ref[...], preferred_element_type=jnp.float32) \
        + b2_ref[...]
    h = jnp.maximum(h, 0.0)
    h = h * (BN_INV * bng_ref[...]) + bnb_ref[...]

    bvec = batch_ref[0, 0, :]
    onehot = jnp.where(
        bvec[:, None] == jax.lax.broadcasted_iota(jnp.int32, (BLK, G), 1),
        1.0, 0.0)
    acc_ref[...] += jax.lax.dot_general(
        onehot, h, (((0,), (0,)), ((), ())),
        preferred_element_type=jnp.float32)

    @pl.when(i == NB - 1)
    def _():
        g = jnp.maximum(
            jax.lax.dot(acc_ref[...], fc1w_ref[...],
                        preferred_element_type=jnp.float32) + fc1b_ref[...],
            0.0)
        out_ref[...] = jax.lax.dot(
            g, fc2w_ref[...], preferred_element_type=jnp.float32) + fc2b_ref[...]


_tc_head = pl.pallas_call(
    _tc_head_body,
    grid=(NB,),
    in_specs=[
        pl.BlockSpec((BLK, D_IN), lambda i: (i, 0)),
        pl.BlockSpec((BLK, D_IN), lambda i: (i, 0)),
        pl.BlockSpec((BLK, D_IN), lambda i: (i, 0)),
        pl.BlockSpec((1, 1, BLK), lambda i: (i, 0, 0)),
        pl.BlockSpec((D_IN, DIM), lambda i: (0, 0)),
        pl.BlockSpec((1, DIM), lambda i: (0, 0)),
        pl.BlockSpec((DIM, DIM), lambda i: (0, 0)),
        pl.BlockSpec((1, DIM), lambda i: (0, 0)),
        pl.BlockSpec((1, DIM), lambda i: (0, 0)),
        pl.BlockSpec((1, DIM), lambda i: (0, 0)),
        pl.BlockSpec((DIM, DIM), lambda i: (0, 0)),
        pl.BlockSpec((1, DIM), lambda i: (0, 0)),
        pl.BlockSpec((DIM, G), lambda i: (0, 0)),
        pl.BlockSpec((1, G), lambda i: (0, 0)),
    ],
    out_specs=pl.BlockSpec((G, G), lambda i: (0, 0)),
    out_shape=jax.ShapeDtypeStruct((G, G), jnp.float32),
    scratch_shapes=[pltpu.VMEM((G, DIM), jnp.float32)],
    compiler_params=pltpu.CompilerParams(
        dimension_semantics=("arbitrary",)),
)


def kernel(x, edge_index, batch, W1, b1, W2, b2, bn_g, bn_b,
           fc1_W, fc1_b, fc2_W, fc2_b):
    src = edge_index[0]
    dst = edge_index[1]
    pad = E_PAD - E
    src_p = jnp.concatenate(
        [src, jnp.zeros((pad,), jnp.int32)]).reshape(NW, CHUNKS_PER_W, CHUNK)
    dst_p = jnp.concatenate(
        [dst, jnp.full((pad,), N, jnp.int32)]).reshape(NW, CHUNKS_PER_W, CHUNK)

    partials = _sc_agg(x, src_p, dst_p)

    xp = jnp.pad(x, ((0, N_TC - N), (0, 0)))
    ap = jnp.pad(partials[0], ((0, N_TC - N), (0, 0)))
    bp = jnp.pad(partials[1], ((0, N_TC - N), (0, 0)))
    batch3 = jnp.pad(batch, (0, N_TC - N), constant_values=G).reshape(NB, 1, BLK)

    fc2p = jnp.pad(fc2_W, ((0, 0), (0, G - 1)))
    fc2bp = jnp.pad(fc2_b, (0, G - 1)).reshape(1, G)

    out = _tc_head(xp, ap, bp, batch3,
                   W1, b1.reshape(1, DIM), W2, b2.reshape(1, DIM),
                   bn_g.reshape(1, DIM), bn_b.reshape(1, DIM),
                   fc1_W, fc1_b.reshape(1, DIM), fc2p, fc2bp)
    return out[:, :1]


# speedup vs baseline: 5.0816x; 5.0816x over previous
import functools

import jax
import jax.numpy as jnp
from jax import lax
from jax.experimental import pallas as pl
from jax.experimental.pallas import tpu as pltpu
from jax.experimental.pallas import tpu_sc as plsc

N = 10000
E = 320000
D_IN = 128
DIM = 256
G = 128

NC = 2
NS = 16
DH = D_IN // NC
CHUNK = 128
CHUNKS_PER_S = 158
E_PAD = NS * CHUNKS_PER_S * CHUNK
N_PAD = N + 8
ROWS_PER_S = 624
TAIL0 = NS * ROWS_PER_S
TAIL = N - TAIL0

BLK = 256
N_TC = 10240
NB = N_TC // BLK
BN_INV = 1.0 / (1.0 + 1e-5) ** 0.5


def _sc_agg_body(x_hbm, src_hbm, dst_hbm, out_hbm,
                 src_v, dst_v, rows0, rows1, acc, sem0, sem1):
    c = lax.axis_index("c")
    s = lax.axis_index("s")
    pltpu.sync_copy(src_hbm.at[c * NS + s], src_v)
    pltpu.sync_copy(dst_hbm.at[s], dst_v)
    row0 = s * ROWS_PER_S
    pltpu.sync_copy(x_hbm.at[pl.ds(c * N + row0, ROWS_PER_S)],
                    acc.at[pl.ds(row0, ROWS_PER_S)])

    @pl.when(s == 0)
    def _():
        pltpu.sync_copy(x_hbm.at[pl.ds(c * N + TAIL0, TAIL)],
                        acc.at[pl.ds(TAIL0, TAIL)])

    plsc.subcore_barrier()

    def pair(i, carry):
        j0 = 2 * i
        j1 = j0 + 1
        g0 = pltpu.async_copy(x_hbm.at[src_v.at[j0]], rows0, sem0)
        g1 = pltpu.async_copy(x_hbm.at[src_v.at[j1]], rows1, sem1)
        g0.wait()
        pltpu.sync_copy(rows0, acc.at[dst_v.at[j0]], add=True)
        g1.wait()
        pltpu.sync_copy(rows1, acc.at[dst_v.at[j1]], add=True)
        return carry

    lax.fori_loop(0, CHUNKS_PER_S // 2, pair, 0)
    plsc.subcore_barrier()
    pltpu.sync_copy(acc.at[pl.ds(row0, ROWS_PER_S)],
                    out_hbm.at[c, pl.ds(row0, ROWS_PER_S)])

    @pl.when(s == 0)
    def _():
        pltpu.sync_copy(acc.at[pl.ds(TAIL0, TAIL)],
                        out_hbm.at[c, pl.ds(TAIL0, TAIL)])


@functools.lru_cache(maxsize=1)
def _sc_agg():
    mesh = plsc.VectorSubcoreMesh(core_axis_name="c", subcore_axis_name="s")
    return pl.kernel(
        _sc_agg_body,
        out_type=jax.ShapeDtypeStruct((NC, N, DH), jnp.float32),
        mesh=mesh,
        scratch_types=[
            pltpu.VMEM((CHUNKS_PER_S, CHUNK), jnp.int32),
            pltpu.VMEM((CHUNKS_PER_S, CHUNK), jnp.int32),
            pltpu.VMEM((CHUNK, DH), jnp.float32),
            pltpu.VMEM((CHUNK, DH), jnp.float32),
            pltpu.VMEM_SHARED((N_PAD, DH), jnp.float32),
            pltpu.SemaphoreType.DMA,
            pltpu.SemaphoreType.DMA,
        ],
        compiler_params=pltpu.CompilerParams(use_tc_tiling_on_sc=False),
    )


def _tc_head_body(a_ref, b_ref, batch_ref,
                  w1_ref, b1_ref, w2_ref, b2_ref, bng_ref, bnb_ref,
                  fc1w_ref, fc1b_ref, fc2w_ref, fc2b_ref,
                  out_ref, acc_ref):
    i = pl.program_id(0)

    @pl.when(i == 0)
    def _():
        acc_ref[...] = jnp.zeros_like(acc_ref)

    h = jnp.concatenate([a_ref[...], b_ref[...]], axis=1)
    h = jnp.maximum(
        jax.lax.dot(h, w1_ref[...], preferred_element_type=jnp.float32)
        + b1_ref[...], 0.0)
    h = jax.lax.dot(h, w2_ref[...], preferred_element_type=jnp.float32) \
        + b2_ref[...]
    h = jnp.maximum(h, 0.0)
    h = h * (BN_INV * bng_ref[...]) + bnb_ref[...]

    bvec = batch_ref[0, 0, :]
    onehot = jnp.where(
        bvec[:, None] == jax.lax.broadcasted_iota(jnp.int32, (BLK, G), 1),
        1.0, 0.0)
    acc_ref[...] += jax.lax.dot_general(
        onehot, h, (((0,), (0,)), ((), ())),
        preferred_element_type=jnp.float32)

    @pl.when(i == NB - 1)
    def _():
        g = jnp.maximum(
            jax.lax.dot(acc_ref[...], fc1w_ref[...],
                        preferred_element_type=jnp.float32) + fc1b_ref[...],
            0.0)
        out_ref[...] = jax.lax.dot(
            g, fc2w_ref[...], preferred_element_type=jnp.float32) + fc2b_ref[...]


_tc_head = pl.pallas_call(
    _tc_head_body,
    grid=(NB,),
    in_specs=[
        pl.BlockSpec((BLK, DH), lambda i: (i, 0)),
        pl.BlockSpec((BLK, DH), lambda i: (i, 0)),
        pl.BlockSpec((1, 1, BLK), lambda i: (i, 0, 0)),
        pl.BlockSpec((D_IN, DIM), lambda i: (0, 0)),
        pl.BlockSpec((1, DIM), lambda i: (0, 0)),
        pl.BlockSpec((DIM, DIM), lambda i: (0, 0)),
        pl.BlockSpec((1, DIM), lambda i: (0, 0)),
        pl.BlockSpec((1, DIM), lambda i: (0, 0)),
        pl.BlockSpec((1, DIM), lambda i: (0, 0)),
        pl.BlockSpec((DIM, DIM), lambda i: (0, 0)),
        pl.BlockSpec((1, DIM), lambda i: (0, 0)),
        pl.BlockSpec((DIM, G), lambda i: (0, 0)),
        pl.BlockSpec((1, G), lambda i: (0, 0)),
    ],
    out_specs=pl.BlockSpec((G, G), lambda i: (0, 0)),
    out_shape=jax.ShapeDtypeStruct((G, G), jnp.float32),
    scratch_shapes=[pltpu.VMEM((G, DIM), jnp.float32)],
    compiler_params=pltpu.CompilerParams(
        dimension_semantics=("arbitrary",)),
)


def kernel(x, edge_index, batch, W1, b1, W2, b2, bn_g, bn_b,
           fc1_W, fc1_b, fc2_W, fc2_b):
    src = edge_index[0]
    dst = edge_index[1]
    pad = E_PAD - E
    src_p = jnp.concatenate([src, jnp.zeros((pad,), jnp.int32)])
    src_p = src_p.reshape(NS, CHUNKS_PER_S, CHUNK)
    src_p = jnp.concatenate([src_p, src_p + N])
    dst_p = jnp.concatenate(
        [dst, jnp.full((pad,), N, jnp.int32)]).reshape(NS, CHUNKS_PER_S, CHUNK)
    x2 = x.reshape(N, NC, DH).transpose(1, 0, 2).reshape(NC * N, DH)

    hio = _sc_agg()(x2, src_p, dst_p)

    ap = jnp.pad(hio[0], ((0, N_TC - N), (0, 0)))
    bp = jnp.pad(hio[1], ((0, N_TC - N), (0, 0)))
    batch3 = jnp.pad(batch, (0, N_TC - N), constant_values=G).reshape(NB, 1, BLK)

    fc2p = jnp.pad(fc2_W, ((0, 0), (0, G - 1)))
    fc2bp = jnp.pad(fc2_b, (0, G - 1)).reshape(1, G)

    out = _tc_head(ap, bp, batch3,
                   W1, b1.reshape(1, DIM), W2, b2.reshape(1, DIM),
                   bn_g.reshape(1, DIM), bn_b.reshape(1, DIM),
                   fc1_W, fc1_b.reshape(1, DIM), fc2p, fc2bp)
    return out[:, :1]
